# baseline (speedup 1.0000x reference)
import jax
import jax.numpy as jnp
from jax import lax
from jax.experimental import pallas as pl

N_TOKENS = 1000
KC = 40


def _onehot_plane(x_ref, o_ref):
    j = pl.program_id(0)
    k0 = pl.program_id(1) * KC
    xb = x_ref[pl.ds(j, 1), :]
    iota = k0 + lax.broadcasted_iota(
        jnp.int32, (KC, x_ref.shape[1]), 0)
    o_ref[0] = (iota == xb).astype(o_ref.dtype)


def kernel(x):
    B, T = x.shape
    xt = x.T
    out_t = pl.pallas_call(
        _onehot_plane,
        grid=(T, N_TOKENS // KC),
        in_specs=[pl.BlockSpec((T, B), lambda j, k: (0, 0))],
        out_specs=pl.BlockSpec((1, KC, B), lambda j, k: (j, k, 0)),
        out_shape=jax.ShapeDtypeStruct((T, N_TOKENS, B), x.dtype),
    )(xt)
    return jnp.transpose(out_t, (2, 0, 1))


# speedup vs baseline: 2.3819x; 2.3819x over previous
import jax
import jax.numpy as jnp
from jax import lax
from jax.experimental import pallas as pl

N_TOKENS = 1000
KC = 200


def _onehot_plane(x_ref, o_ref):
    j = pl.program_id(0)
    k0 = pl.program_id(1) * KC
    xb = x_ref[pl.ds(j, 1), :]
    iota = k0 + lax.broadcasted_iota(
        jnp.int32, (KC, x_ref.shape[1]), 0)
    o_ref[0] = (iota == xb).astype(o_ref.dtype)


def kernel(x):
    B, T = x.shape
    xt = x.T
    out_t = pl.pallas_call(
        _onehot_plane,
        grid=(T, N_TOKENS // KC),
        in_specs=[pl.BlockSpec((T, B), lambda j, k: (0, 0))],
        out_specs=pl.BlockSpec((1, KC, B), lambda j, k: (j, k, 0)),
        out_shape=jax.ShapeDtypeStruct((T, N_TOKENS, B), x.dtype),
    )(xt)
    return jnp.transpose(out_t, (2, 0, 1))
